# baseline (speedup 1.0000x reference)
import functools

import jax
import jax.numpy as jnp
from jax import lax
from jax.experimental import pallas as pl
from jax.experimental.pallas import tpu as pltpu
from jax.experimental.pallas import tpu_sc as plsc

_B, _H, _W, _K, _P, _C = 2, 224, 224, 8, 100000, 128
_N = _B * _H * _W
_HT = _H // 8
_WT = 2
_NPW = _N // 32
_NSTG = 4
_NCH = _NPW // 16 // _NSTG
_NBUF = 5
_CHUNK = 16


def _take16(v, idx):
    return v.at[idx].get(mode="promise_in_bounds")


def _transpose16(vs, iota16):
    vs = list(vs)
    for d in (8, 4, 2, 1):
        rot_p = (iota16 - d) & 15
        rot_m = (iota16 + d) & 15
        m = (iota16 & d) != 0
        for i in range(16):
            if i & d:
                continue
            j = i | d
            a, b = vs[i], vs[j]
            vs[i] = jnp.where(m, _take16(b, rot_p), a)
            vs[j] = jnp.where(m, b, _take16(a, rot_m))
    return vs


def _sc_body(feat_hbm, idx_hbm, dists_hbm, depth_hbm, out_hbm, pm_hbm,
             idx_v, dist_v, rows_v, out_v, dbuf, slab, sems, rsems, fsems):
    iota16 = lax.broadcasted_iota(jnp.int32, (16,), 0)
    cid = lax.axis_index("c")
    sid = lax.axis_index("s")

    wid = cid * 16 + sid
    pix0 = pl.multiple_of(wid * _NPW, 8)

    def fire(g, slot):
        goff = pl.multiple_of(g * (_CHUNK * _K), 8)
        pltpu.make_async_copy(
            feat_hbm.at[idx_v.at[pl.ds(goff, _CHUNK * _K)]],
            rows_v.at[slot], sems.at[slot],
        ).start()

    def compute(gg, g, slot):
        goff = pl.multiple_of(g * (_CHUNK * _K), 8)
        winvs = []
        for j in range(8):
            d = dist_v[pl.ds(goff + j * 16, 16)]
            w = 1.0 - d
            s = w
            for sh in (1, 2, 4):
                s = s + _take16(s, iota16 ^ sh)
            winvs.append(w / jnp.maximum(s, 1e-10))

        for p in range(16):
            wv = winvs[p // 2]
            base = (p % 2) * 8
            accs = None
            for k in range(8):
                ws = _take16(wv, jnp.zeros_like(iota16) + (base + k))
                r = p * 8 + k
                terms = [ws * rows_v[slot, r, pl.ds(cb * 16, 16)]
                         for cb in range(8)]
                accs = terms if accs is None else [a + t
                                                   for a, t in zip(accs, terms)]
            for cb in range(8):
                out_v[p, pl.ds(cb * 16, 16)] = accs[cb]

        prow = pl.multiple_of(pix0 + gg * _CHUNK, 8)
        pltpu.sync_copy(out_v, pm_hbm.at[pl.ds(prow, _CHUNK)])

    for stg in range(_NSTG):
        off = pl.multiple_of((pix0 + stg * (_NCH * _CHUNK)) * _K, 8)
        pltpu.sync_copy(idx_hbm.at[pl.ds(off, _NCH * _CHUNK * _K)], idx_v)
        pltpu.sync_copy(dists_hbm.at[pl.ds(off, _NCH * _CHUNK * _K)], dist_v)

        for slot in range(_NBUF):
            fire(slot, slot)

        def chunk_body(g, carry, stg=stg):
            slot = lax.rem(g, _NBUF)
            pltpu.make_async_copy(
                feat_hbm.at[idx_v.at[pl.ds(0, _CHUNK * _K)]],
                rows_v.at[slot], sems.at[slot],
            ).wait()
            compute(stg * _NCH + g, g, slot)

            @pl.when(g + _NBUF < _NCH)
            def _():
                fire(g + _NBUF, slot)

            return carry

        lax.fori_loop(0, _NCH, chunk_body, 0)

    def depth_tile(td, carry):
        t = sid + 16 * td

        @pl.when(t < _HT * _WT)
        def _():
            t_ht = lax.div(t, _WT)
            t_wt = lax.rem(t, _WT)
            for wt_s, npx in ((0, 128), (1, 96)):
                @pl.when(t_wt == wt_s)
                def _copy_rows():
                    for r in range(8):
                        src = pl.multiple_of(
                            cid * (_H * _W) + (t_ht * 8 + r) * _W
                            + wt_s * 128, 8)
                        pltpu.sync_copy(depth_hbm.at[pl.ds(src, npx)],
                                        dbuf.at[r, pl.ds(0, npx)])
            pltpu.sync_copy(
                dbuf,
                out_hbm.at[cid * (_C + 1) + _C,
                           pl.ds(pl.multiple_of(t_ht * 8, 8), 8),
                           pl.ds(pl.multiple_of(t_wt * 128, 128), 128)],
            )
        return carry

    lax.fori_loop(0, (_HT * _WT + 15) // 16, depth_tile, 0)

    plsc.subcore_barrier()

    tm = lax.div(sid, 8)
    hl = lax.rem(sid, 8)
    tbuf = rows_v.at[2]
    ntile = _HT * _WT // 2

    def tile_coords(t):
        u = tm * ntile + t
        return lax.div(u, _WT), lax.rem(u, _WT)

    def fire_read(t):
        t_ht, t_wt = tile_coords(t)
        prow = pl.multiple_of(
            cid * (_H * _W) + (t_ht * 8 + hl) * _W + t_wt * 128, 8)
        pltpu.make_async_copy(
            pm_hbm.at[pl.ds(prow, 128)],
            rows_v.at[lax.rem(t, 2)], rsems.at[lax.rem(t, 2)],
        ).start()

    def flush_dma(buf, t_ht, t_wt):
        return pltpu.make_async_copy(
            slab.at[buf, tm],
            out_hbm.at[pl.ds(cid * (_C + 1), _C),
                       pl.ds(pl.multiple_of(t_ht * 8, 8), 8),
                       pl.ds(pl.multiple_of(t_wt * 128, 128), 128)],
            fsems.at[buf],
        )

    fire_read(0)

    def tile_body(t, carry):
        u_ht, u_wt = tile_coords(t)
        npb = 8 - 2 * u_wt
        rslot = lax.rem(t, 2)
        buf = lax.rem(t, 2)
        rbuf = rows_v.at[rslot]

        pltpu.make_async_copy(
            pm_hbm.at[pl.ds(pl.multiple_of(0, 8), 128)],
            rows_v.at[rslot], rsems.at[rslot],
        ).wait()

        @pl.when(t + 1 < ntile)
        def _prefetch():
            fire_read(t + 1)

        def pb_body(pb, carry2):
            pcol = pl.multiple_of(pb * 16, 8)
            for cb in range(8):
                cols = _transpose16(
                    [rbuf[pb * 16 + i, pl.ds(cb * 16, 16)]
                     for i in range(16)], iota16)
                for ci in range(16):
                    tbuf[cb * 16 + ci, pl.ds(pcol, 16)] = cols[ci]
            return carry2

        lax.fori_loop(0, npb, pb_body, 0)

        pltpu.sync_copy(tbuf, slab.at[buf, tm, :, hl])

        @pl.when(jnp.logical_and(hl == 0, t >= 1))
        def _drain():
            flush_dma(lax.rem(t + 1, 2), u_ht, u_wt).wait()

        plsc.subcore_barrier()

        @pl.when(hl == 0)
        def _flush():
            flush_dma(buf, u_ht, u_wt).start()

        return carry

    lax.fori_loop(0, ntile, tile_body, 0)

    @pl.when(hl == 0)
    def _drain_tail():
        t_ht, t_wt = tile_coords(ntile - 1)
        flush_dma((ntile - 1) % 2, t_ht, t_wt).wait()


_sc_render = functools.partial(
    pl.kernel,
    mesh=plsc.VectorSubcoreMesh(core_axis_name="c", subcore_axis_name="s"),
    out_type=[
        jax.ShapeDtypeStruct((_B * (_C + 1), _H, _W), jnp.float32),
        jax.ShapeDtypeStruct((_N + 32, _C), jnp.float32),
    ],
    scratch_types=[
        pltpu.VMEM((_NCH * _CHUNK * _K,), jnp.int32),
        pltpu.VMEM((_NCH * _CHUNK * _K,), jnp.float32),
        pltpu.VMEM((_NBUF, _CHUNK * _K, _C), jnp.float32),
        pltpu.VMEM((_CHUNK, _C), jnp.float32),
        pltpu.VMEM((8, 128), jnp.float32),
        pltpu.VMEM_SHARED((2, 2, _C, 8, 128), jnp.float32),
        pltpu.SemaphoreType.DMA((_NBUF,)),
        pltpu.SemaphoreType.DMA((2,)),
        pltpu.SemaphoreType.DMA((2,)),
    ],
)(_sc_body)


def _depth_body(z_ref, o_ref):
    x = z_ref[...]
    m = x == -1.0
    ne = jnp.sum(m.astype(jnp.float32), axis=1, keepdims=True)
    dsum = jnp.sum(x, axis=1, keepdims=True) + ne
    mean = dsum / (float(_H * _W) - ne)
    o_ref[...] = jnp.where(m, -1.0, x - mean)


def _depth_normalize(depth_raw):
    return pl.pallas_call(
        _depth_body,
        out_shape=jax.ShapeDtypeStruct((_B, _H * _W), jnp.float32),
    )(depth_raw)


def kernel(idx, zbuf, dists, features):
    idx_flat = idx.astype(jnp.int32).reshape(_N * _K)
    d_flat = dists.reshape(_N * _K)
    depth_raw = zbuf[..., 0].reshape(_B, _H * _W)
    depth_n = _depth_normalize(depth_raw).reshape(_N)
    out3, _ = _sc_render(features, idx_flat, d_flat, depth_n)
    out4 = out3.reshape(_B, _C + 1, _H, _W)
    return jnp.transpose(out4, (0, 2, 3, 1))


# speedup vs baseline: 1.0419x; 1.0419x over previous
import functools

import jax
import jax.numpy as jnp
from jax import lax
from jax.experimental import pallas as pl
from jax.experimental.pallas import tpu as pltpu
from jax.experimental.pallas import tpu_sc as plsc

_B, _H, _W, _K, _P, _C = 2, 224, 224, 8, 100000, 128
_N = _B * _H * _W
_HT = _H // 8
_WT = 2
_NPW = _N // 32
_NCH = _NPW // 16 // 2
_NBUF = 4
_CHUNK = 16


def _take16(v, idx):
    return v.at[idx].get(mode="promise_in_bounds")


def _transpose16(vs, iota16):
    vs = list(vs)
    for d in (8, 4, 2, 1):
        rot_p = (iota16 - d) & 15
        rot_m = (iota16 + d) & 15
        m = (iota16 & d) != 0
        for i in range(16):
            if i & d:
                continue
            j = i | d
            a, b = vs[i], vs[j]
            vs[i] = jnp.where(m, _take16(b, rot_p), a)
            vs[j] = jnp.where(m, b, _take16(a, rot_m))
    return vs


def _sc_body(feat_hbm, idx_hbm, dists_hbm, depth_hbm, out_hbm, pm_hbm,
             idx_v, dist_v, rows_v, out_v, dbuf, slab, sems, rsems, fsems):
    iota16 = lax.broadcasted_iota(jnp.int32, (16,), 0)
    cid = lax.axis_index("c")
    sid = lax.axis_index("s")

    wid = cid * 16 + sid
    pix0 = pl.multiple_of(wid * _NPW, 8)

    def fire(g, slot):
        goff = pl.multiple_of(g * (_CHUNK * _K), 8)
        pltpu.make_async_copy(
            feat_hbm.at[idx_v.at[pl.ds(goff, _CHUNK * _K)]],
            rows_v.at[slot], sems.at[slot],
        ).start()

    def compute(gg, g, slot):
        goff = pl.multiple_of(g * (_CHUNK * _K), 8)
        winvs = []
        for j in range(8):
            d = dist_v[pl.ds(goff + j * 16, 16)]
            w = 1.0 - d
            s = w
            for sh in (1, 2, 4):
                s = s + _take16(s, iota16 ^ sh)
            winvs.append(w / jnp.maximum(s, 1e-10))

        for p in range(16):
            wv = winvs[p // 2]
            base = (p % 2) * 8
            accs = None
            for k in range(8):
                ws = _take16(wv, jnp.zeros_like(iota16) + (base + k))
                r = p * 8 + k
                terms = [ws * rows_v[slot, r, pl.ds(cb * 16, 16)]
                         for cb in range(8)]
                accs = terms if accs is None else [a + t
                                                   for a, t in zip(accs, terms)]
            for cb in range(8):
                out_v[p, pl.ds(cb * 16, 16)] = accs[cb]

        prow = pl.multiple_of(pix0 + gg * _CHUNK, 8)
        pltpu.sync_copy(out_v, pm_hbm.at[pl.ds(prow, _CHUNK)])

    for half in range(2):
        off = pl.multiple_of((pix0 + half * (_NCH * _CHUNK)) * _K, 8)
        pltpu.sync_copy(idx_hbm.at[pl.ds(off, _NCH * _CHUNK * _K)], idx_v)
        pltpu.sync_copy(dists_hbm.at[pl.ds(off, _NCH * _CHUNK * _K)], dist_v)

        for slot in range(_NBUF):
            fire(slot, slot)

        def chunk_body(g, carry, half=half):
            slot = lax.rem(g, _NBUF)
            pltpu.make_async_copy(
                feat_hbm.at[idx_v.at[pl.ds(0, _CHUNK * _K)]],
                rows_v.at[slot], sems.at[slot],
            ).wait()
            compute(half * _NCH + g, g, slot)

            @pl.when(g + _NBUF < _NCH)
            def _():
                fire(g + _NBUF, slot)

            return carry

        lax.fori_loop(0, _NCH, chunk_body, 0)

    def depth_tile(td, carry):
        t = sid + 16 * td

        @pl.when(t < _HT * _WT)
        def _():
            t_ht = lax.div(t, _WT)
            t_wt = lax.rem(t, _WT)
            for wt_s, npx in ((0, 128), (1, 96)):
                @pl.when(t_wt == wt_s)
                def _copy_rows():
                    for r in range(8):
                        src = pl.multiple_of(
                            cid * (_H * _W) + (t_ht * 8 + r) * _W
                            + wt_s * 128, 8)
                        pltpu.sync_copy(depth_hbm.at[pl.ds(src, npx)],
                                        dbuf.at[r, pl.ds(0, npx)])
            pltpu.sync_copy(
                dbuf,
                out_hbm.at[cid * (_C + 1) + _C,
                           pl.ds(pl.multiple_of(t_ht * 8, 8), 8),
                           pl.ds(pl.multiple_of(t_wt * 128, 128), 128)],
            )
        return carry

    lax.fori_loop(0, (_HT * _WT + 15) // 16, depth_tile, 0)

    plsc.subcore_barrier()

    tm = lax.div(sid, 8)
    hl = lax.rem(sid, 8)
    tbuf = rows_v.at[2]
    ntile = _HT * _WT // 2

    def tile_coords(t):
        u = tm * ntile + t
        return lax.div(u, _WT), lax.rem(u, _WT)

    def fire_read(t):
        t_ht, t_wt = tile_coords(t)
        prow = pl.multiple_of(
            cid * (_H * _W) + (t_ht * 8 + hl) * _W + t_wt * 128, 8)
        pltpu.make_async_copy(
            pm_hbm.at[pl.ds(prow, 128)],
            rows_v.at[lax.rem(t, 2)], rsems.at[lax.rem(t, 2)],
        ).start()

    def flush_dma(buf, t_ht, t_wt):
        return pltpu.make_async_copy(
            slab.at[buf, tm],
            out_hbm.at[pl.ds(cid * (_C + 1), _C),
                       pl.ds(pl.multiple_of(t_ht * 8, 8), 8),
                       pl.ds(pl.multiple_of(t_wt * 128, 128), 128)],
            fsems.at[buf],
        )

    fire_read(0)

    def tile_body(t, carry):
        u_ht, u_wt = tile_coords(t)
        npb = 8 - 2 * u_wt
        rslot = lax.rem(t, 2)
        buf = lax.rem(t, 2)
        rbuf = rows_v.at[rslot]

        pltpu.make_async_copy(
            pm_hbm.at[pl.ds(pl.multiple_of(0, 8), 128)],
            rows_v.at[rslot], rsems.at[rslot],
        ).wait()

        @pl.when(t + 1 < ntile)
        def _prefetch():
            fire_read(t + 1)

        def pb_body(pb, carry2):
            pcol = pl.multiple_of(pb * 16, 8)
            for cb in range(8):
                cols = _transpose16(
                    [rbuf[pb * 16 + i, pl.ds(cb * 16, 16)]
                     for i in range(16)], iota16)
                for ci in range(16):
                    tbuf[cb * 16 + ci, pl.ds(pcol, 16)] = cols[ci]
            return carry2

        lax.fori_loop(0, npb, pb_body, 0)

        pltpu.sync_copy(tbuf, slab.at[buf, tm, :, hl])

        @pl.when(jnp.logical_and(hl == 0, t >= 1))
        def _drain():
            flush_dma(lax.rem(t + 1, 2), u_ht, u_wt).wait()

        plsc.subcore_barrier()

        @pl.when(hl == 0)
        def _flush():
            flush_dma(buf, u_ht, u_wt).start()

        return carry

    lax.fori_loop(0, ntile, tile_body, 0)

    @pl.when(hl == 0)
    def _drain_tail():
        t_ht, t_wt = tile_coords(ntile - 1)
        flush_dma((ntile - 1) % 2, t_ht, t_wt).wait()


_sc_render = functools.partial(
    pl.kernel,
    mesh=plsc.VectorSubcoreMesh(core_axis_name="c", subcore_axis_name="s"),
    out_type=[
        jax.ShapeDtypeStruct((_B * (_C + 1), _H, _W), jnp.float32),
        jax.ShapeDtypeStruct((_N + 32, _C), jnp.float32),
    ],
    scratch_types=[
        pltpu.VMEM((_NCH * _CHUNK * _K,), jnp.int32),
        pltpu.VMEM((_NCH * _CHUNK * _K,), jnp.float32),
        pltpu.VMEM((_NBUF, _CHUNK * _K, _C), jnp.float32),
        pltpu.VMEM((_CHUNK, _C), jnp.float32),
        pltpu.VMEM((8, 128), jnp.float32),
        pltpu.VMEM_SHARED((2, 2, _C, 8, 128), jnp.float32),
        pltpu.SemaphoreType.DMA((_NBUF,)),
        pltpu.SemaphoreType.DMA((2,)),
        pltpu.SemaphoreType.DMA((2,)),
    ],
)(_sc_body)


def _depth_body(z_ref, o_ref):
    x = z_ref[...]
    m = x == -1.0
    ne = jnp.sum(m.astype(jnp.float32), axis=1, keepdims=True)
    dsum = jnp.sum(x, axis=1, keepdims=True) + ne
    mean = dsum / (float(_H * _W) - ne)
    o_ref[...] = jnp.where(m, -1.0, x - mean)


def _depth_normalize(depth_raw):
    return pl.pallas_call(
        _depth_body,
        out_shape=jax.ShapeDtypeStruct((_B, _H * _W), jnp.float32),
    )(depth_raw)


def kernel(idx, zbuf, dists, features):
    idx_flat = idx.astype(jnp.int32).reshape(_N * _K)
    d_flat = dists.reshape(_N * _K)
    depth_raw = zbuf[..., 0].reshape(_B, _H * _W)
    depth_n = _depth_normalize(depth_raw).reshape(_N)
    out3, _ = _sc_render(features, idx_flat, d_flat, depth_n)
    out4 = out3.reshape(_B, _C + 1, _H, _W)
    return jnp.transpose(out4, (0, 2, 3, 1))
